# baseline (speedup 1.0000x reference)
import functools

import jax
import jax.numpy as jnp
from jax import lax
from jax.experimental import pallas as pl
from jax.experimental.pallas import tpu as pltpu
from jax.experimental.pallas import tpu_sc as plsc

_CHUNK = 64
_GROUP = 8


def _sc_half(x_aug, ei, attr, zs, ze, n_pad, per_w, start):
  _, d_aug = x_aug.shape
  da = attr.shape[1]
  info = plsc.get_sparse_core_info()
  ns = info.num_subcores
  chunk = _CHUNK
  group = _GROUP
  gc = group * chunk
  n_groups = per_w // gc
  rows_per_tile = n_pad // ns

  mesh = plsc.VectorSubcoreMesh(
      core_axis_name="c", subcore_axis_name="s", num_cores=1)

  @functools.partial(
      pl.kernel,
      out_type=(
          jax.ShapeDtypeStruct((n_pad, d_aug), jnp.float32),
          jax.ShapeDtypeStruct((n_pad, da), jnp.float32),
      ),
      mesh=mesh,
      compiler_params=pltpu.CompilerParams(use_tc_tiling_on_sc=False),
      scratch_types=[
          pltpu.VMEM_SHARED((n_pad, d_aug), jnp.float32),
          pltpu.VMEM_SHARED((n_pad, da), jnp.float32),
          pltpu.VMEM((2, gc), jnp.int32),
          [pltpu.VMEM((chunk, d_aug), jnp.float32) for _ in range(2)],
          pltpu.VMEM((gc, da), jnp.float32),
          [pltpu.SemaphoreType.DMA for _ in range(2)],
      ],
  )
  def k(x_hbm, ei_hbm, attr_hbm, zs_hbm, ze_hbm,
        s_out, e_out,
        sh_s, sh_e, ebuf, xbuf, abuf, sem):
    sid = lax.axis_index("s")
    r0 = sid * rows_per_tile
    pltpu.sync_copy(zs_hbm.at[pl.ds(r0, rows_per_tile)],
                    sh_s.at[pl.ds(r0, rows_per_tile)])
    pltpu.sync_copy(ze_hbm.at[pl.ds(r0, rows_per_tile)],
                    sh_e.at[pl.ds(r0, rows_per_tile)])
    plsc.subcore_barrier()

    def gather(c, p):
      return pltpu.make_async_copy(
          x_hbm.at[ebuf.at[1, pl.ds(c * chunk, chunk)]], xbuf[p], sem[p])

    def body(g, carry):
      base = start + sid * per_w + g * gc
      pltpu.sync_copy(ei_hbm.at[:, pl.ds(base, gc)], ebuf)
      pltpu.sync_copy(attr_hbm.at[pl.ds(base, gc)], abuf)
      gather(0, 0).start()
      for c in range(group):
        p = c % 2
        rows = ebuf.at[0, pl.ds(c * chunk, chunk)]
        gather(c, p).wait()
        if c + 1 < group:
          gather(c + 1, 1 - p).start()
        pltpu.sync_copy(xbuf[p], sh_s.at[rows], add=True)
        pltpu.sync_copy(abuf.at[pl.ds(c * chunk, chunk)],
                        sh_e.at[rows], add=True)
      return carry

    lax.fori_loop(0, n_groups, body, 0)

    plsc.subcore_barrier()
    pltpu.sync_copy(sh_s.at[pl.ds(r0, rows_per_tile)],
                    s_out.at[pl.ds(r0, rows_per_tile)])
    pltpu.sync_copy(sh_e.at[pl.ds(r0, rows_per_tile)],
                    e_out.at[pl.ds(r0, rows_per_tile)])

  return k(x_aug, ei, attr, zs, ze)


def _sc_segment_sums(x, edge_index, edge_attr):
  n, d = x.shape
  ne = edge_index.shape[1]
  da = edge_attr.shape[1]
  d_aug = d + 16
  info = plsc.get_sparse_core_info()
  ns = info.num_subcores
  nhalf = 2 * ns
  chunk = _CHUNK
  gc = _GROUP * chunk
  unit = nhalf * gc
  ne_pad = -(-ne // unit) * unit
  per_w = ne_pad // nhalf
  n_pad = -(-n // (8 * ns)) * (8 * ns)

  pad = ne_pad - ne
  if pad:
    ei_pad = jnp.stack([jnp.full((pad,), n, jnp.int32),
                        jnp.zeros((pad,), jnp.int32)])
    edge_index = jnp.concatenate([edge_index, ei_pad], axis=1)
    edge_attr = jnp.concatenate(
        [edge_attr, jnp.zeros((pad, da), jnp.float32)], axis=0)

  x_aug = jnp.concatenate(
      [x, jnp.ones((n, 1), jnp.float32),
       jnp.zeros((n, d_aug - d - 1), jnp.float32)], axis=1)

  zs = jnp.zeros((n_pad, d_aug), jnp.float32)
  ze = jnp.zeros((n_pad, da), jnp.float32)

  half = ne_pad // 2
  s0, e0 = _sc_half(x_aug, edge_index, edge_attr, zs, ze, n_pad, per_w, 0)
  s1, e1 = _sc_half(x_aug, edge_index, edge_attr, zs, ze, n_pad, per_w, half)
  return (jnp.stack([s0, s1]), jnp.stack([e0, e1]))


def _tc_finish(s2, e2, w, b):
  nc, n, d_aug = s2.shape
  da = e2.shape[2]
  d = d_aug - 16
  blk = 1264
  grid = n // blk
  b2 = b.reshape(1, d)

  def body(s_ref, e_ref, w_ref, b_ref, o_ref):
    s = s_ref[0, :, 0:d] + s_ref[1, :, 0:d]
    cnt = s_ref[0, :, d:d + 1] + s_ref[1, :, d:d + 1]
    e = e_ref[0] + e_ref[1]
    acc = jnp.dot(s, w_ref[0:d, :], preferred_element_type=jnp.float32)
    acc = acc + jnp.dot(e, w_ref[d:, :], preferred_element_type=jnp.float32)
    o_ref[...] = acc / jnp.maximum(cnt, 1.0) + b_ref[...]

  return pl.pallas_call(
      body,
      grid=(grid,),
      in_specs=[
          pl.BlockSpec((nc, blk, d_aug), lambda i: (0, i, 0)),
          pl.BlockSpec((nc, blk, da), lambda i: (0, i, 0)),
          pl.BlockSpec((d + da, d), lambda i: (0, 0)),
          pl.BlockSpec((1, d), lambda i: (0, 0)),
      ],
      out_specs=pl.BlockSpec((blk, d), lambda i: (i, 0)),
      out_shape=jax.ShapeDtypeStruct((n, d), jnp.float32),
  )(s2, e2, w, b2)


def kernel(x, edge_index, edge_attr, W, b):
  s2, e2 = _sc_segment_sums(x, edge_index, edge_attr)
  return _tc_finish(s2, e2, W, b)[: x.shape[0]]
